# baseline (speedup 1.0000x reference)
import jax
import jax.numpy as jnp
from jax import lax
from jax.experimental import pallas as pl
from jax.experimental.pallas import tpu as pltpu

_ROWS = 32
_CHUNK = 12800


def _col_chunks(n):
    c0 = 0
    while c0 < n:
        w = min(_CHUNK, n - c0)
        yield c0, w
        c0 += w


def _stats_body(qhat_ref, x_ref, xmax_ref, z_ref, vstar_ref, jstar_ref,
                full_ref):
    qhat = qhat_ref[0]
    x = x_ref[...]
    R, N = x.shape

    xmax = jnp.max(x, axis=1, keepdims=True)
    e = jnp.exp(x - xmax)
    z = jnp.sum(e, axis=1, keepdims=True)
    s = e / z

    rmax = jnp.float32(1.0) / z
    hi0 = lax.bitcast_convert_type(rmax, jnp.int32) + 1
    lo_v = jnp.maximum((jnp.float32(0.999) - qhat) / jnp.float32(2 * N), 0.0)
    lo0 = jnp.full((R, 1), lax.bitcast_convert_type(lo_v, jnp.int32))

    def val_step(carry):
        lo, hi = carry
        mid = (lo + hi) >> 1
        t = lax.bitcast_convert_type(mid, jnp.float32)
        ssum = jnp.sum(jnp.where(s >= t, s, 0.0), axis=1, keepdims=True)
        pred = ssum > qhat
        return jnp.where(pred, mid, lo), jnp.where(pred, hi, mid)

    carry = lax.fori_loop(0, 26, lambda _, c: val_step(c), (lo0, hi0))
    lo, _ = lax.while_loop(lambda c: jnp.any(c[1] - c[0] > 1), val_step, carry)
    vstar = lax.bitcast_convert_type(lo, jnp.float32)

    gt = s > vstar
    eq = s == vstar
    s_gt = jnp.sum(jnp.where(gt, s, 0.0), axis=1, keepdims=True)
    m_eq = jnp.sum(eq.astype(jnp.int32), axis=1, keepdims=True)

    vsafe = jnp.where(vstar > 0.0, vstar, 1.0)
    r = jnp.floor((qhat - s_gt) / vsafe).astype(jnp.int32) + 1
    r = jnp.clip(r, 1, jnp.maximum(m_eq, 1))

    def count_le(j):
        cnt = jnp.zeros((R, 1), jnp.int32)
        for c0, w in _col_chunks(N):
            idx_c = lax.broadcasted_iota(jnp.int32, (R, w), 1) + c0
            cnt = cnt + jnp.sum((eq[:, c0:c0 + w] & (idx_c <= j)).astype(
                jnp.int32), axis=1, keepdims=True)
        return cnt

    def tie_idx_general():
        def idx_step(_, carry):
            lo_j, hi_j = carry
            mid_j = (lo_j + hi_j) >> 1
            ge = count_le(mid_j) >= r
            return jnp.where(ge, lo_j, mid_j), jnp.where(ge, mid_j, hi_j)

        lo_j = jnp.full((R, 1), -1, jnp.int32)
        hi_j = jnp.full((R, 1), N - 1, jnp.int32)
        _, hi_j = lax.fori_loop(0, 17, idx_step, (lo_j, hi_j))
        return hi_j

    def tie_idx_single():
        best = jnp.full((R, 1), N, jnp.int32)
        for c0, w in _col_chunks(N):
            idx_c = lax.broadcasted_iota(jnp.int32, (R, w), 1) + c0
            sub = jnp.min(jnp.where(eq[:, c0:c0 + w], idx_c, N), axis=1,
                          keepdims=True)
            best = jnp.minimum(best, sub)
        return best

    jstar = lax.cond(jnp.any(m_eq > 1), tie_idx_general, tie_idx_single)

    full_row = (lo == 0) | (qhat >= 1.0)

    xmax_ref[...] = xmax
    z_ref[...] = z
    vstar_ref[...] = vstar
    jstar_ref[...] = jstar
    full_ref[...] = full_row.astype(jnp.int32)


def _mask_body(x_ref, xmax_ref, z_ref, vstar_ref, jstar_ref, full_ref,
               mask_ref):
    x = x_ref[...]
    R, W = x.shape
    j = pl.program_id(1)
    s = jnp.exp(x - xmax_ref[...]) / z_ref[...]
    vstar = vstar_ref[...]
    idx = lax.broadcasted_iota(jnp.int32, (R, W), 1) + j * W
    mask = (s > vstar) | ((s == vstar) & (idx <= jstar_ref[...]))
    mask_ref[...] = mask | (full_ref[...] != 0)


def kernel(logits, Qhat):
    b, n = logits.shape
    qhat1 = jnp.asarray(Qhat, jnp.float32).reshape(1)
    row_spec = pl.BlockSpec((_ROWS, 1), lambda i: (i, 0))
    stat = lambda dt: jax.ShapeDtypeStruct((b, 1), dt)
    xmax, z, vstar, jstar, full = pl.pallas_call(
        _stats_body,
        grid=(b // _ROWS,),
        in_specs=[
            pl.BlockSpec(memory_space=pltpu.SMEM),
            pl.BlockSpec((_ROWS, n), lambda i: (i, 0)),
        ],
        out_specs=[row_spec] * 5,
        out_shape=[stat(jnp.float32), stat(jnp.float32), stat(jnp.float32),
                   stat(jnp.int32), stat(jnp.int32)],
    )(qhat1, logits)

    row2 = pl.BlockSpec((_ROWS, 1), lambda i, j: (i, 0))
    mask = pl.pallas_call(
        _mask_body,
        grid=(b // _ROWS, pl.cdiv(n, _CHUNK)),
        in_specs=[pl.BlockSpec((_ROWS, _CHUNK), lambda i, j: (i, j))]
                 + [row2] * 5,
        out_specs=pl.BlockSpec((_ROWS, _CHUNK), lambda i, j: (i, j)),
        out_shape=jax.ShapeDtypeStruct((b, n), jnp.bool_),
    )(logits, xmax, z, vstar, jstar, full)
    return (logits, mask)


# speedup vs baseline: 1.1295x; 1.1295x over previous
import jax
import jax.numpy as jnp
from jax import lax
from jax.experimental import pallas as pl
from jax.experimental.pallas import tpu as pltpu

_ROWS = 32
_CHUNK = 12800


def _col_chunks(n):
    c0 = 0
    while c0 < n:
        w = min(_CHUNK, n - c0)
        yield c0, w
        c0 += w


def _sum_chunked(arr):
    parts = [jnp.sum(arr[:, c0:c0 + w], axis=1, keepdims=True)
             for c0, w in _col_chunks(arr.shape[1])]
    while len(parts) > 1:
        parts = [a + b for a, b in zip(parts[::2], parts[1::2])] + (
            [parts[-1]] if len(parts) % 2 else [])
    return parts[0]


def _stats_body(qhat_ref, x_ref, xmax_ref, z_ref, vstar_ref, jstar_ref,
                full_ref):
    qhat = qhat_ref[0]
    x = x_ref[...]
    R, N = x.shape

    xmax = jnp.max(x, axis=1, keepdims=True)
    e = jnp.exp(x - xmax)
    z = _sum_chunked(e)
    s = e / z

    rmax = jnp.float32(1.0) / z
    hi0 = lax.bitcast_convert_type(rmax, jnp.int32) + 1
    lo_v = jnp.maximum((jnp.float32(0.999) - qhat) / jnp.float32(2 * N), 0.0)
    lo0 = jnp.full((R, 1), lax.bitcast_convert_type(lo_v, jnp.int32))

    def val_step(carry):
        lo, hi = carry
        mid = (lo + hi) >> 1
        t = lax.bitcast_convert_type(mid, jnp.float32)
        ssum = _sum_chunked(jnp.where(s >= t, s, 0.0))
        pred = ssum > qhat
        return jnp.where(pred, mid, lo), jnp.where(pred, hi, mid)

    carry = lax.fori_loop(0, 26, lambda _, c: val_step(c), (lo0, hi0))
    lo, _ = lax.while_loop(lambda c: jnp.any(c[1] - c[0] > 1), val_step, carry)
    vstar = lax.bitcast_convert_type(lo, jnp.float32)

    gt = s > vstar
    eq = s == vstar
    s_gt = _sum_chunked(jnp.where(gt, s, 0.0))
    m_eq = _sum_chunked(eq.astype(jnp.int32))

    vsafe = jnp.where(vstar > 0.0, vstar, 1.0)
    r = jnp.floor((qhat - s_gt) / vsafe).astype(jnp.int32) + 1
    r = jnp.clip(r, 1, jnp.maximum(m_eq, 1))

    def count_le(j):
        cnt = jnp.zeros((R, 1), jnp.int32)
        for c0, w in _col_chunks(N):
            idx_c = lax.broadcasted_iota(jnp.int32, (R, w), 1) + c0
            cnt = cnt + jnp.sum((eq[:, c0:c0 + w] & (idx_c <= j)).astype(
                jnp.int32), axis=1, keepdims=True)
        return cnt

    def tie_idx_general():
        def idx_step(_, carry):
            lo_j, hi_j = carry
            mid_j = (lo_j + hi_j) >> 1
            ge = count_le(mid_j) >= r
            return jnp.where(ge, lo_j, mid_j), jnp.where(ge, mid_j, hi_j)

        lo_j = jnp.full((R, 1), -1, jnp.int32)
        hi_j = jnp.full((R, 1), N - 1, jnp.int32)
        _, hi_j = lax.fori_loop(0, 17, idx_step, (lo_j, hi_j))
        return hi_j

    def tie_idx_single():
        best = jnp.full((R, 1), N, jnp.int32)
        for c0, w in _col_chunks(N):
            idx_c = lax.broadcasted_iota(jnp.int32, (R, w), 1) + c0
            sub = jnp.min(jnp.where(eq[:, c0:c0 + w], idx_c, N), axis=1,
                          keepdims=True)
            best = jnp.minimum(best, sub)
        return best

    jstar = lax.cond(jnp.any(m_eq > 1), tie_idx_general, tie_idx_single)

    full_row = (lo == 0) | (qhat >= 1.0)

    xmax_ref[...] = xmax
    z_ref[...] = z
    vstar_ref[...] = vstar
    jstar_ref[...] = jstar
    full_ref[...] = full_row.astype(jnp.int32)


def _mask_body(x_ref, xmax_ref, z_ref, vstar_ref, jstar_ref, full_ref,
               mask_ref):
    x = x_ref[...]
    R, W = x.shape
    j = pl.program_id(1)
    s = jnp.exp(x - xmax_ref[...]) / z_ref[...]
    vstar = vstar_ref[...]
    idx = lax.broadcasted_iota(jnp.int32, (R, W), 1) + j * W
    mask = (s > vstar) | ((s == vstar) & (idx <= jstar_ref[...]))
    mask_ref[...] = mask | (full_ref[...] != 0)


def kernel(logits, Qhat):
    b, n = logits.shape
    qhat1 = jnp.asarray(Qhat, jnp.float32).reshape(1)
    row_spec = pl.BlockSpec((_ROWS, 1), lambda i: (i, 0))
    stat = lambda dt: jax.ShapeDtypeStruct((b, 1), dt)
    xmax, z, vstar, jstar, full = pl.pallas_call(
        _stats_body,
        grid=(b // _ROWS,),
        in_specs=[
            pl.BlockSpec(memory_space=pltpu.SMEM),
            pl.BlockSpec((_ROWS, n), lambda i: (i, 0)),
        ],
        out_specs=[row_spec] * 5,
        out_shape=[stat(jnp.float32), stat(jnp.float32), stat(jnp.float32),
                   stat(jnp.int32), stat(jnp.int32)],
    )(qhat1, logits)

    row2 = pl.BlockSpec((_ROWS, 1), lambda i, j: (i, 0))
    mask = pl.pallas_call(
        _mask_body,
        grid=(b // _ROWS, pl.cdiv(n, _CHUNK)),
        in_specs=[pl.BlockSpec((_ROWS, _CHUNK), lambda i, j: (i, j))]
                 + [row2] * 5,
        out_specs=pl.BlockSpec((_ROWS, _CHUNK), lambda i, j: (i, j)),
        out_shape=jax.ShapeDtypeStruct((b, n), jnp.bool_),
    )(logits, xmax, z, vstar, jstar, full)
    return (logits, mask)
